# Initial kernel scaffold; baseline (speedup 1.0000x reference)
import functools

import jax
import jax.numpy as jnp
from jax import lax
from jax.experimental import pallas as pl
from jax.experimental.pallas import tpu as pltpu
from jax.experimental.pallas import tpu_sc as plsc

_KE = 14.399645
_CUTON = 2.5
_SW_CUTOFF = 7.5
_LR_CUTOFF = 10.0
_CUTOFF = 10.0
_HALF_KE = 0.5 * _KE
_INV_SW_WIDTH = 1.0 / (_SW_CUTOFF - _CUTON)
_INV_LR = 1.0 / _LR_CUTOFF
_INV_LR2 = 1.0 / (_LR_CUTOFF * _LR_CUTOFF)
_INV_CUT = 1.0 / _CUTOFF
_INV_A_CONST = 1.0 / (0.8854 * 0.529177)

_NC = 2
_NS = 16
_NW = _NC * _NS
_LANES = 16
_SUBLEN = 128
_CHUNK = 1024
_NSUB = _CHUNK // _SUBLEN


def _rsqrt(x):
    i = lax.bitcast_convert_type(x, jnp.int32)
    i = jnp.int32(0x5F3759DF) - lax.shift_right_logical(i, 1)
    y = lax.bitcast_convert_type(i, jnp.float32)
    xh = 0.5 * x
    for _ in range(3):
        y = y * (1.5 - xh * y * y)
    return y


def _pair_energy(xi, yi_, zi_, qi, c6i, zfi, z23i,
                 xj, yj_, zj_, qj, c6j, zfj, z23j):
    dx = xj - xi
    dy = yj_ - yi_
    dz = zj_ - zi_
    r2 = dx * dx + dy * dy + dz * dz
    u = _rsqrt(r2)
    r = r2 * u
    damped = _rsqrt(r2 + 1.0)
    xs = jnp.clip((r - _CUTON) * _INV_SW_WIDTH, 0.0, 1.0)
    s = (xs * xs * xs) * (xs * (6.0 * xs - 15.0) + 10.0)
    coul = damped + s * (u - damped)
    in_lr = r < _LR_CUTOFF
    shifted = jnp.where(in_lr, coul + r * _INV_LR2 - 2.0 * _INV_LR, 0.0)
    e_c = (qi * qj) * shifted
    xc = r * _INV_CUT
    fc = jnp.where(in_lr,
                   1.0 - (xc * xc * xc) * (xc * (6.0 * xc - 15.0) + 10.0),
                   0.0)
    c6p = c6i * c6j + 1e-12
    c6ij = c6p * _rsqrt(c6p)
    r6 = r2 * r2 * r2
    w6 = _rsqrt(r6 + 1.0)
    e_d = (-0.5) * c6ij * (w6 * w6) * fc
    inv_a = (z23i + z23j + 1e-9) * _INV_A_CONST
    xz = r * inv_a
    phi = (0.18175 * jnp.exp(-3.19980 * xz)
           + 0.50986 * jnp.exp(-0.94229 * xz)
           + 0.28022 * jnp.exp(-0.40290 * xz)
           + 0.02817 * jnp.exp(-0.20162 * xz))
    e_z = (zfi * zfj) * u * phi * fc
    return _HALF_KE * (e_c + e_z) + e_d


def _make_sc_kernel(nt, nchunk):
    rows_per_tile = nt // _NS
    mesh = plsc.VectorSubcoreMesh(core_axis_name="c", subcore_axis_name="s",
                                  num_cores=_NC, num_subcores=_NS)

    @functools.partial(
        pl.kernel,
        out_type=jax.ShapeDtypeStruct((_NC, _NS, rows_per_tile), jnp.float32),
        mesh=mesh,
        scratch_types=[
            pltpu.VMEM((_NSUB, _SUBLEN), jnp.int32),
            pltpu.VMEM((_NSUB, _SUBLEN), jnp.int32),
            pltpu.VMEM((_CHUNK, 8), jnp.float32),
            pltpu.VMEM((_CHUNK, 8), jnp.float32),
            pltpu.VMEM((_CHUNK,), jnp.float32),
            pltpu.VMEM((nt // _NS,), jnp.float32),
            pltpu.VMEM_SHARED((nt, 8), jnp.float32),
            pltpu.VMEM_SHARED((nt,), jnp.float32),
            pltpu.SemaphoreType.DMA,
            pltpu.SemaphoreType.DMA,
        ],
    )
    def sc_kernel(table_hbm, idxi_hbm, idxj_hbm, out_hbm,
                  idxi_v, idxj_v, rows_i, rows_j, evals, zbuf,
                  table_sp, acc_sp, sem_i, sem_j):
        cid = lax.axis_index("c")
        tid = lax.axis_index("s")

        r0 = tid * rows_per_tile
        pltpu.sync_copy(table_hbm.at[pl.ds(r0, rows_per_tile)],
                        table_sp.at[pl.ds(r0, rows_per_tile)])

        def zero_body(i, _):
            zbuf[pl.ds(i * _LANES, _LANES)] = jnp.zeros((_LANES,), jnp.float32)
            return 0
        lax.fori_loop(0, rows_per_tile // _LANES, zero_body, 0)
        pltpu.sync_copy(zbuf, acc_sp.at[pl.ds(r0, rows_per_tile)])
        plsc.subcore_barrier()

        wid = cid * _NS + tid
        base_row = wid * (nchunk * _NSUB)
        lanes = lax.iota(jnp.int32, _LANES)

        def chunk_body(c, _):
            row0 = base_row + c * _NSUB
            pltpu.sync_copy(idxi_hbm.at[pl.ds(row0, _NSUB)], idxi_v)
            pltpu.sync_copy(idxj_hbm.at[pl.ds(row0, _NSUB)], idxj_v)
            cps = []
            for sub in range(_NSUB):
                dst = pl.ds(sub * _SUBLEN, _SUBLEN)
                cps.append(pltpu.async_copy(table_sp.at[idxi_v.at[sub]],
                                            rows_i.at[dst], sem_i))
                cps.append(pltpu.async_copy(table_sp.at[idxj_v.at[sub]],
                                            rows_j.at[dst], sem_j))
            for cp in cps:
                cp.wait()

            def pair_body(k, _):
                row16 = k * _LANES + lanes

                def col(ref, ci):
                    return plsc.load_gather(
                        ref, [row16, jnp.full((_LANES,), ci, jnp.int32)])

                e16 = _pair_energy(
                    col(rows_i, 0), col(rows_i, 1), col(rows_i, 2),
                    col(rows_i, 3), col(rows_i, 4), col(rows_i, 5),
                    col(rows_i, 6),
                    col(rows_j, 0), col(rows_j, 1), col(rows_j, 2),
                    col(rows_j, 3), col(rows_j, 4), col(rows_j, 5),
                    col(rows_j, 6))
                evals[pl.ds(k * _LANES, _LANES)] = e16
                return 0

            lax.fori_loop(0, _CHUNK // _LANES, pair_body, 0)

            for sub in range(_NSUB):
                pltpu.sync_copy(evals.at[pl.ds(sub * _SUBLEN, _SUBLEN)],
                                acc_sp.at[idxi_v.at[sub]], add=True)
            return 0

        lax.fori_loop(0, nchunk, chunk_body, 0)
        plsc.subcore_barrier()
        pltpu.sync_copy(acc_sp.at[pl.ds(r0, rows_per_tile)],
                        out_hbm.at[cid, tid])

    return sc_kernel


def _combine_body(p_ref, y_ref, o_ref):
    o_ref[...] = p_ref[0] + p_ref[1] + y_ref[...]


def kernel(yi, R, partial_charges, c6_table, Z, idx_m, idx_i, idx_j):
    n = Z.shape[0]
    p = idx_i.shape[0]

    nt = ((n + 1 + 2047) // 2048) * 2048
    pairs_per_sweep = _NW * _CHUNK
    nchunk = (p + pairs_per_sweep - 1) // pairs_per_sweep
    p_pad = nchunk * pairs_per_sweep

    Z = Z.astype(jnp.int32)
    zf = Z.astype(jnp.float32)
    c6 = jax.nn.softplus(c6_table.astype(jnp.float32))
    table = jnp.stack(
        [R[:, 0], R[:, 1], R[:, 2],
         partial_charges.astype(jnp.float32),
         c6[Z], zf, zf ** 0.23, jnp.zeros((n,), jnp.float32)], axis=1)
    table = jnp.pad(table, ((0, nt - n), (0, 0)))

    ii = jnp.pad(idx_i.astype(jnp.int32), (0, p_pad - p), constant_values=n)
    jj = jnp.pad(idx_j.astype(jnp.int32), (0, p_pad - p), constant_values=n)
    ii = ii.reshape(p_pad // _SUBLEN, _SUBLEN)
    jj = jj.reshape(p_pad // _SUBLEN, _SUBLEN)

    parts = _make_sc_kernel(nt, nchunk)(table, ii, jj)
    parts = parts.reshape(_NC, nt // 128, 128)

    yi0 = jnp.pad(yi[:, 0].astype(jnp.float32), (0, nt - n))
    yi0 = yi0.reshape(nt // 128, 128)

    total = pl.pallas_call(
        _combine_body,
        out_shape=jax.ShapeDtypeStruct((nt // 128, 128), jnp.float32),
    )(parts, yi0)

    return total.reshape(nt)[:n][:, None]


# speedup vs baseline: 140.4007x; 140.4007x over previous
import functools

import jax
import jax.numpy as jnp
from jax import lax
from jax.experimental import pallas as pl
from jax.experimental.pallas import tpu as pltpu
from jax.experimental.pallas import tpu_sc as plsc

_KE = 14.399645
_CUTON = 2.5
_SW_CUTOFF = 7.5
_LR_CUTOFF = 10.0
_CUTOFF = 10.0
_HALF_KE = 0.5 * _KE
_INV_SW_WIDTH = 1.0 / (_SW_CUTOFF - _CUTON)
_INV_LR = 1.0 / _LR_CUTOFF
_INV_LR2 = 1.0 / (_LR_CUTOFF * _LR_CUTOFF)
_INV_CUT = 1.0 / _CUTOFF
_INV_A_CONST = 1.0 / (0.8854 * 0.529177)

_NC = 2
_NS = 16
_NW = _NC * _NS
_LANES = 16
_SUBLEN = 128
_CHUNK = 1024
_NSUB = _CHUNK // _SUBLEN


def _rsqrt(x):
    i = lax.bitcast_convert_type(x, jnp.int32)
    i = jnp.int32(0x5F3759DF) - lax.shift_right_logical(i, 1)
    y = lax.bitcast_convert_type(i, jnp.float32)
    xh = 0.5 * x
    for _ in range(3):
        y = y * (1.5 - xh * y * y)
    return y


def _pair_energy(xi, yi_, zi_, qi, c6i, zfi, z23i,
                 xj, yj_, zj_, qj, c6j, zfj, z23j):
    dx = xj - xi
    dy = yj_ - yi_
    dz = zj_ - zi_
    r2 = dx * dx + dy * dy + dz * dz
    u = _rsqrt(r2)
    r = r2 * u
    damped = _rsqrt(r2 + 1.0)
    xs = jnp.clip((r - _CUTON) * _INV_SW_WIDTH, 0.0, 1.0)
    s = (xs * xs * xs) * (xs * (6.0 * xs - 15.0) + 10.0)
    coul = damped + s * (u - damped)
    in_lr = r < _LR_CUTOFF
    shifted = jnp.where(in_lr, coul + r * _INV_LR2 - 2.0 * _INV_LR, 0.0)
    e_c = (qi * qj) * shifted
    xc = r * _INV_CUT
    fc = jnp.where(in_lr,
                   1.0 - (xc * xc * xc) * (xc * (6.0 * xc - 15.0) + 10.0),
                   0.0)
    c6p = c6i * c6j + 1e-12
    c6ij = c6p * _rsqrt(c6p)
    r6 = r2 * r2 * r2
    w6 = _rsqrt(r6 + 1.0)
    e_d = (-0.5) * c6ij * (w6 * w6) * fc
    inv_a = (z23i + z23j + 1e-9) * _INV_A_CONST
    xz = r * inv_a
    phi = (0.18175 * jnp.exp(-3.19980 * xz)
           + 0.50986 * jnp.exp(-0.94229 * xz)
           + 0.28022 * jnp.exp(-0.40290 * xz)
           + 0.02817 * jnp.exp(-0.20162 * xz))
    e_z = (zfi * zfj) * u * phi * fc
    return _HALF_KE * (e_c + e_z) + e_d


def _make_sc_kernel(nt, nchunk):
    rows_per_tile = nt // _NS
    mesh = plsc.VectorSubcoreMesh(core_axis_name="c", subcore_axis_name="s",
                                  num_cores=_NC, num_subcores=_NS)

    @functools.partial(
        pl.kernel,
        out_type=jax.ShapeDtypeStruct((_NC, _NS, rows_per_tile), jnp.float32),
        mesh=mesh,
        scratch_types=[
            pltpu.VMEM((_NSUB, _SUBLEN), jnp.int32),
            pltpu.VMEM((_NSUB, _SUBLEN), jnp.int32),
            pltpu.VMEM((_CHUNK, 8), jnp.float32),
            pltpu.VMEM((_CHUNK, 8), jnp.float32),
            pltpu.VMEM((_CHUNK,), jnp.float32),
            pltpu.VMEM((nt // _NS,), jnp.float32),
            pltpu.VMEM_SHARED((nt,), jnp.float32),
            pltpu.SemaphoreType.DMA,
            pltpu.SemaphoreType.DMA,
        ],
        compiler_params=pltpu.CompilerParams(needs_layout_passes=False,
                                             use_tc_tiling_on_sc=False),
    )
    def sc_kernel(table_hbm, idxi_hbm, idxj_hbm, out_hbm,
                  idxi_v, idxj_v, rows_i, rows_j, evals, zbuf,
                  acc_sp, sem_i, sem_j):
        cid = lax.axis_index("c")
        tid = lax.axis_index("s")

        r0 = tid * rows_per_tile

        def zero_body(i, _):
            zbuf[pl.ds(i * _LANES, _LANES)] = jnp.zeros((_LANES,), jnp.float32)
            return 0
        lax.fori_loop(0, rows_per_tile // _LANES, zero_body, 0)
        pltpu.sync_copy(zbuf, acc_sp.at[pl.ds(r0, rows_per_tile)])
        plsc.subcore_barrier()

        wid = cid * _NS + tid
        base_row = wid * (nchunk * _NSUB)
        lanes = lax.iota(jnp.int32, _LANES)

        def chunk_body(c, _):
            row0 = base_row + c * _NSUB
            pltpu.sync_copy(idxi_hbm.at[pl.ds(row0, _NSUB)], idxi_v)
            pltpu.sync_copy(idxj_hbm.at[pl.ds(row0, _NSUB)], idxj_v)
            cps = []
            for sub in range(_NSUB):
                dst = pl.ds(sub * _SUBLEN, _SUBLEN)
                cps.append(pltpu.async_copy(table_hbm.at[idxi_v.at[sub]],
                                            rows_i.at[dst], sem_i))
                cps.append(pltpu.async_copy(table_hbm.at[idxj_v.at[sub]],
                                            rows_j.at[dst], sem_j))
            for cp in cps:
                cp.wait()

            def pair_body(k, _):
                row16 = k * _LANES + lanes

                def col(ref, ci):
                    return plsc.load_gather(
                        ref, [row16, jnp.full((_LANES,), ci, jnp.int32)])

                e16 = _pair_energy(
                    col(rows_i, 0), col(rows_i, 1), col(rows_i, 2),
                    col(rows_i, 3), col(rows_i, 4), col(rows_i, 5),
                    col(rows_i, 6),
                    col(rows_j, 0), col(rows_j, 1), col(rows_j, 2),
                    col(rows_j, 3), col(rows_j, 4), col(rows_j, 5),
                    col(rows_j, 6))
                evals[pl.ds(k * _LANES, _LANES)] = e16
                return 0

            lax.fori_loop(0, _CHUNK // _LANES, pair_body, 0)

            for sub in range(_NSUB):
                pltpu.sync_copy(evals.at[pl.ds(sub * _SUBLEN, _SUBLEN)],
                                acc_sp.at[idxi_v.at[sub]], add=True)
            return 0

        lax.fori_loop(0, nchunk, chunk_body, 0)
        plsc.subcore_barrier()
        pltpu.sync_copy(acc_sp.at[pl.ds(r0, rows_per_tile)],
                        out_hbm.at[cid, tid])

    return sc_kernel


def _combine_body(p_ref, y_ref, o_ref):
    o_ref[...] = p_ref[0] + p_ref[1] + y_ref[...]


def kernel(yi, R, partial_charges, c6_table, Z, idx_m, idx_i, idx_j):
    n = Z.shape[0]
    p = idx_i.shape[0]

    nt = ((n + 1 + 2047) // 2048) * 2048
    pairs_per_sweep = _NW * _CHUNK
    nchunk = (p + pairs_per_sweep - 1) // pairs_per_sweep
    p_pad = nchunk * pairs_per_sweep

    Z = Z.astype(jnp.int32)
    zf = Z.astype(jnp.float32)
    c6 = jax.nn.softplus(c6_table.astype(jnp.float32))
    table = jnp.stack(
        [R[:, 0], R[:, 1], R[:, 2],
         partial_charges.astype(jnp.float32),
         c6[Z], zf, zf ** 0.23, jnp.zeros((n,), jnp.float32)], axis=1)
    table = jnp.pad(table, ((0, nt - n), (0, 0)))

    ii = jnp.pad(idx_i.astype(jnp.int32), (0, p_pad - p), constant_values=n)
    jj = jnp.pad(idx_j.astype(jnp.int32), (0, p_pad - p), constant_values=n)
    ii = ii.reshape(p_pad // _SUBLEN, _SUBLEN)
    jj = jj.reshape(p_pad // _SUBLEN, _SUBLEN)

    parts = _make_sc_kernel(nt, nchunk)(table, ii, jj)
    parts = parts.reshape(_NC, nt // 128, 128)

    yi0 = jnp.pad(yi[:, 0].astype(jnp.float32), (0, nt - n))
    yi0 = yi0.reshape(nt // 128, 128)

    total = pl.pallas_call(
        _combine_body,
        out_shape=jax.ShapeDtypeStruct((nt // 128, 128), jnp.float32),
    )(parts, yi0)

    return total.reshape(nt)[:n][:, None]


# speedup vs baseline: 164.5382x; 1.1719x over previous
import functools

import jax
import jax.numpy as jnp
from jax import lax
from jax.experimental import pallas as pl
from jax.experimental.pallas import tpu as pltpu
from jax.experimental.pallas import tpu_sc as plsc

_KE = 14.399645
_CUTON = 2.5
_SW_CUTOFF = 7.5
_LR_CUTOFF = 10.0
_CUTOFF = 10.0
_HALF_KE = 0.5 * _KE
_INV_SW_WIDTH = 1.0 / (_SW_CUTOFF - _CUTON)
_INV_LR = 1.0 / _LR_CUTOFF
_INV_LR2 = 1.0 / (_LR_CUTOFF * _LR_CUTOFF)
_INV_CUT = 1.0 / _CUTOFF
_INV_A_CONST = 1.0 / (0.8854 * 0.529177)

_NC = 2
_NS = 16
_NW = _NC * _NS
_LANES = 16
_SUBLEN = 128
_CHUNK = 1024
_NSUB = _CHUNK // _SUBLEN


def _rsqrt(x):
    i = lax.bitcast_convert_type(x, jnp.int32)
    i = jnp.int32(0x5F3759DF) - lax.shift_right_logical(i, 1)
    y = lax.bitcast_convert_type(i, jnp.float32)
    xh = 0.5 * x
    for _ in range(3):
        y = y * (1.5 - xh * y * y)
    return y


def _pair_energy(xi, yi_, zi_, qi, c6i, zfi, z23i,
                 xj, yj_, zj_, qj, c6j, zfj, z23j):
    dx = xj - xi
    dy = yj_ - yi_
    dz = zj_ - zi_
    r2 = dx * dx + dy * dy + dz * dz
    u = _rsqrt(r2)
    r = r2 * u
    damped = _rsqrt(r2 + 1.0)
    xs = jnp.clip((r - _CUTON) * _INV_SW_WIDTH, 0.0, 1.0)
    s = (xs * xs * xs) * (xs * (6.0 * xs - 15.0) + 10.0)
    coul = damped + s * (u - damped)
    in_lr = r < _LR_CUTOFF
    shifted = jnp.where(in_lr, coul + r * _INV_LR2 - 2.0 * _INV_LR, 0.0)
    e_c = (qi * qj) * shifted
    xc = r * _INV_CUT
    fc = jnp.where(in_lr,
                   1.0 - (xc * xc * xc) * (xc * (6.0 * xc - 15.0) + 10.0),
                   0.0)
    c6p = c6i * c6j + 1e-12
    c6ij = c6p * _rsqrt(c6p)
    r6 = r2 * r2 * r2
    w6 = _rsqrt(r6 + 1.0)
    e_d = (-0.5) * c6ij * (w6 * w6) * fc
    inv_a = (z23i + z23j + 1e-9) * _INV_A_CONST
    xz = r * inv_a
    phi = (0.18175 * jnp.exp(-3.19980 * xz)
           + 0.50986 * jnp.exp(-0.94229 * xz)
           + 0.28022 * jnp.exp(-0.40290 * xz)
           + 0.02817 * jnp.exp(-0.20162 * xz))
    e_z = (zfi * zfj) * u * phi * fc
    return _HALF_KE * (e_c + e_z) + e_d


def _make_sc_kernel(nt, nchunk):
    rows_per_tile = nt // _NS
    mesh = plsc.VectorSubcoreMesh(core_axis_name="c", subcore_axis_name="s",
                                  num_cores=_NC, num_subcores=_NS)

    scr = []
    for _ in range(2):
        scr += [
            pltpu.VMEM((_NSUB, _SUBLEN), jnp.int32),
            pltpu.VMEM((_NSUB, _SUBLEN), jnp.int32),
            pltpu.VMEM((_CHUNK, 8), jnp.float32),
            pltpu.VMEM((_CHUNK, 8), jnp.float32),
            pltpu.VMEM((_CHUNK,), jnp.float32),
            pltpu.SemaphoreType.DMA,
        ]
    scr += [
        pltpu.VMEM((nt // _NS,), jnp.float32),
        pltpu.VMEM_SHARED((nt,), jnp.float32),
    ]

    @functools.partial(
        pl.kernel,
        out_type=jax.ShapeDtypeStruct((_NC, _NS, rows_per_tile), jnp.float32),
        mesh=mesh,
        scratch_types=scr,
        compiler_params=pltpu.CompilerParams(needs_layout_passes=False,
                                             use_tc_tiling_on_sc=False),
    )
    def sc_kernel(table_hbm, idxi_hbm, idxj_hbm, out_hbm, *bufs):
        (idxi_a, idxj_a, ri_a, rj_a, ev_a, sem_a,
         idxi_b, idxj_b, ri_b, rj_b, ev_b, sem_b,
         zbuf, acc_sp) = bufs
        buf = [(idxi_a, idxj_a, ri_a, rj_a, ev_a, sem_a),
               (idxi_b, idxj_b, ri_b, rj_b, ev_b, sem_b)]
        cid = lax.axis_index("c")
        tid = lax.axis_index("s")

        r0 = tid * rows_per_tile

        def zero_body(i, _):
            zbuf[pl.ds(i * _LANES, _LANES)] = jnp.zeros((_LANES,), jnp.float32)
            return 0
        lax.fori_loop(0, rows_per_tile // _LANES, zero_body, 0)
        pltpu.sync_copy(zbuf, acc_sp.at[pl.ds(r0, rows_per_tile)])
        plsc.subcore_barrier()

        wid = cid * _NS + tid
        base_row = wid * (nchunk * _NSUB)
        lanes = lax.iota(jnp.int32, _LANES)

        def load_idx(b, c):
            row0 = base_row + c * _NSUB
            pltpu.sync_copy(idxi_hbm.at[pl.ds(row0, _NSUB)], buf[b][0])
            pltpu.sync_copy(idxj_hbm.at[pl.ds(row0, _NSUB)], buf[b][1])

        def gather_cps(b, issue):
            idxi_v, idxj_v, rows_i, rows_j, _, sem = buf[b]
            fn = pltpu.async_copy if issue else (
                lambda s, d, m: pltpu.make_async_copy(s, d, m))
            cps = []
            for sub in range(_NSUB):
                dst = pl.ds(sub * _SUBLEN, _SUBLEN)
                cps.append(fn(table_hbm.at[idxi_v.at[sub]],
                              rows_i.at[dst], sem))
                cps.append(fn(table_hbm.at[idxj_v.at[sub]],
                              rows_j.at[dst], sem))
            return cps

        def compute_scatter(b):
            idxi_v, _, rows_i, rows_j, evals, _ = buf[b]

            def pair_body(k, _):
                row16 = k * _LANES + lanes

                def col(ref, ci):
                    return plsc.load_gather(
                        ref, [row16, jnp.full((_LANES,), ci, jnp.int32)])

                e16 = _pair_energy(
                    col(rows_i, 0), col(rows_i, 1), col(rows_i, 2),
                    col(rows_i, 3), col(rows_i, 4), col(rows_i, 5),
                    col(rows_i, 6),
                    col(rows_j, 0), col(rows_j, 1), col(rows_j, 2),
                    col(rows_j, 3), col(rows_j, 4), col(rows_j, 5),
                    col(rows_j, 6))
                evals[pl.ds(k * _LANES, _LANES)] = e16
                return 0

            lax.fori_loop(0, _CHUNK // _LANES, pair_body, 0)

            for sub in range(_NSUB):
                pltpu.sync_copy(evals.at[pl.ds(sub * _SUBLEN, _SUBLEN)],
                                acc_sp.at[idxi_v.at[sub]], add=True)

        load_idx(0, 0)
        gather_cps(0, True)

        def sched_body(g, _):
            load_idx(1, 2 * g + 1)
            gather_cps(1, True)
            for cp in gather_cps(0, False):
                cp.wait()
            compute_scatter(0)

            @pl.when(g < nchunk // 2 - 1)
            def _():
                load_idx(0, 2 * g + 2)
                gather_cps(0, True)

            for cp in gather_cps(1, False):
                cp.wait()
            compute_scatter(1)
            return 0

        lax.fori_loop(0, nchunk // 2, sched_body, 0)
        plsc.subcore_barrier()
        pltpu.sync_copy(acc_sp.at[pl.ds(r0, rows_per_tile)],
                        out_hbm.at[cid, tid])

    return sc_kernel


def _combine_body(p_ref, y_ref, o_ref):
    o_ref[...] = p_ref[0] + p_ref[1] + y_ref[...]


def kernel(yi, R, partial_charges, c6_table, Z, idx_m, idx_i, idx_j):
    n = Z.shape[0]
    p = idx_i.shape[0]

    nt = ((n + 1 + 2047) // 2048) * 2048
    pairs_per_sweep = _NW * _CHUNK
    nchunk = (p + pairs_per_sweep - 1) // pairs_per_sweep
    nchunk += nchunk % 2
    p_pad = nchunk * pairs_per_sweep

    Z = Z.astype(jnp.int32)
    zf = Z.astype(jnp.float32)
    c6 = jax.nn.softplus(c6_table.astype(jnp.float32))
    table = jnp.stack(
        [R[:, 0], R[:, 1], R[:, 2],
         partial_charges.astype(jnp.float32),
         c6[Z], zf, zf ** 0.23, jnp.zeros((n,), jnp.float32)], axis=1)
    table = jnp.pad(table, ((0, nt - n), (0, 0)))

    ii = jnp.pad(idx_i.astype(jnp.int32), (0, p_pad - p), constant_values=n)
    jj = jnp.pad(idx_j.astype(jnp.int32), (0, p_pad - p), constant_values=n)
    ii = ii.reshape(p_pad // _SUBLEN, _SUBLEN)
    jj = jj.reshape(p_pad // _SUBLEN, _SUBLEN)

    parts = _make_sc_kernel(nt, nchunk)(table, ii, jj)
    parts = parts.reshape(_NC, nt // 128, 128)

    yi0 = jnp.pad(yi[:, 0].astype(jnp.float32), (0, nt - n))
    yi0 = yi0.reshape(nt // 128, 128)

    total = pl.pallas_call(
        _combine_body,
        out_shape=jax.ShapeDtypeStruct((nt // 128, 128), jnp.float32),
    )(parts, yi0)

    return total.reshape(nt)[:n][:, None]


# speedup vs baseline: 179.6516x; 1.0919x over previous
import functools

import jax
import jax.numpy as jnp
from jax import lax
from jax.experimental import pallas as pl
from jax.experimental.pallas import tpu as pltpu
from jax.experimental.pallas import tpu_sc as plsc

_KE = 14.399645
_CUTON = 2.5
_SW_CUTOFF = 7.5
_LR_CUTOFF = 10.0
_CUTOFF = 10.0
_HALF_KE = 0.5 * _KE
_INV_SW_WIDTH = 1.0 / (_SW_CUTOFF - _CUTON)
_INV_LR = 1.0 / _LR_CUTOFF
_INV_LR2 = 1.0 / (_LR_CUTOFF * _LR_CUTOFF)
_INV_CUT = 1.0 / _CUTOFF
_INV_A_CONST = 1.0 / (0.8854 * 0.529177)

_NC = 2
_NS = 16
_NW = _NC * _NS
_LANES = 16
_SUBLEN = 128
_CHUNK = 1024
_NSUB = _CHUNK // _SUBLEN


def _rsqrt(x):
    i = lax.bitcast_convert_type(x, jnp.int32)
    i = jnp.int32(0x5F3759DF) - lax.shift_right_logical(i, 1)
    y = lax.bitcast_convert_type(i, jnp.float32)
    xh = 0.5 * x
    for _ in range(2):
        y = y * (1.5 - xh * y * y)
    return y


def _pair_energy(xi, yi_, zi_, qi, c6i, zfi, z23i,
                 xj, yj_, zj_, qj, c6j, zfj, z23j):
    dx = xj - xi
    dy = yj_ - yi_
    dz = zj_ - zi_
    r2 = dx * dx + dy * dy + dz * dz
    u = _rsqrt(r2)
    r = r2 * u
    damped = _rsqrt(r2 + 1.0)
    xs = jnp.clip((r - _CUTON) * _INV_SW_WIDTH, 0.0, 1.0)
    s = (xs * xs * xs) * (xs * (6.0 * xs - 15.0) + 10.0)
    coul = damped + s * (u - damped)
    in_lr = r < _LR_CUTOFF
    shifted = jnp.where(in_lr, coul + r * _INV_LR2 - 2.0 * _INV_LR, 0.0)
    e_c = (qi * qj) * shifted
    xc = r * _INV_CUT
    fc = jnp.where(in_lr,
                   1.0 - (xc * xc * xc) * (xc * (6.0 * xc - 15.0) + 10.0),
                   0.0)
    c6p = c6i * c6j + 1e-12
    c6ij = c6p * _rsqrt(c6p)
    r6 = r2 * r2 * r2
    w6 = _rsqrt(r6 + 1.0)
    e_d = (-0.5) * c6ij * (w6 * w6) * fc
    inv_a = (z23i + z23j + 1e-9) * _INV_A_CONST
    xz = r * inv_a
    phi = (0.18175 * jnp.exp(-3.19980 * xz)
           + 0.50986 * jnp.exp(-0.94229 * xz)
           + 0.28022 * jnp.exp(-0.40290 * xz)
           + 0.02817 * jnp.exp(-0.20162 * xz))
    e_z = (zfi * zfj) * u * phi * fc
    return _HALF_KE * (e_c + e_z) + e_d


def _make_sc_kernel(nt, nchunk):
    rows_per_tile = nt // _NS
    mesh = plsc.VectorSubcoreMesh(core_axis_name="c", subcore_axis_name="s",
                                  num_cores=_NC, num_subcores=_NS)

    scr = []
    for _ in range(2):
        scr += [
            pltpu.VMEM((_NSUB, _SUBLEN), jnp.int32),
            pltpu.VMEM((_NSUB, _SUBLEN), jnp.int32),
            pltpu.VMEM((_CHUNK, 8), jnp.float32),
            pltpu.VMEM((_CHUNK, 8), jnp.float32),
            pltpu.VMEM((_CHUNK,), jnp.float32),
            pltpu.SemaphoreType.DMA,
            pltpu.SemaphoreType.DMA,
        ]
    scr += [
        pltpu.VMEM((nt // _NS,), jnp.float32),
        pltpu.VMEM_SHARED((nt,), jnp.float32),
    ]

    @functools.partial(
        pl.kernel,
        out_type=jax.ShapeDtypeStruct((_NC, _NS, rows_per_tile), jnp.float32),
        mesh=mesh,
        scratch_types=scr,
        compiler_params=pltpu.CompilerParams(needs_layout_passes=False,
                                             use_tc_tiling_on_sc=False),
    )
    def sc_kernel(table_hbm, idxi_hbm, idxj_hbm, out_hbm, *bufs):
        (idxi_a, idxj_a, ri_a, rj_a, ev_a, sem_a, ssem_a,
         idxi_b, idxj_b, ri_b, rj_b, ev_b, sem_b, ssem_b,
         zbuf, acc_sp) = bufs
        buf = [(idxi_a, idxj_a, ri_a, rj_a, ev_a, sem_a, ssem_a),
               (idxi_b, idxj_b, ri_b, rj_b, ev_b, sem_b, ssem_b)]
        cid = lax.axis_index("c")
        tid = lax.axis_index("s")

        r0 = tid * rows_per_tile

        def zero_body(i, _):
            zbuf[pl.ds(i * _LANES, _LANES)] = jnp.zeros((_LANES,), jnp.float32)
            return 0
        lax.fori_loop(0, rows_per_tile // _LANES, zero_body, 0)
        pltpu.sync_copy(zbuf, acc_sp.at[pl.ds(r0, rows_per_tile)])
        plsc.subcore_barrier()

        wid = cid * _NS + tid
        base_row = wid * (nchunk * _NSUB)
        lanes = lax.iota(jnp.int32, _LANES)

        def load_idx(b, c):
            row0 = base_row + c * _NSUB
            sem = buf[b][5]
            c1 = pltpu.async_copy(idxi_hbm.at[pl.ds(row0, _NSUB)],
                                  buf[b][0], sem)
            c2 = pltpu.async_copy(idxj_hbm.at[pl.ds(row0, _NSUB)],
                                  buf[b][1], sem)
            c1.wait()
            c2.wait()

        def gather_cps(b, issue):
            idxi_v, idxj_v, rows_i, rows_j, _, sem, _ssem = buf[b]
            fn = pltpu.async_copy if issue else (
                lambda s, d, m: pltpu.make_async_copy(s, d, m))
            cps = []
            for sub in range(_NSUB):
                dst = pl.ds(sub * _SUBLEN, _SUBLEN)
                cps.append(fn(table_hbm.at[idxi_v.at[sub]],
                              rows_i.at[dst], sem))
                cps.append(fn(table_hbm.at[idxj_v.at[sub]],
                              rows_j.at[dst], sem))
            return cps

        def compute_scatter(b):
            idxi_v, _, rows_i, rows_j, evals, _sem, ssem = buf[b]

            def pair_body(k, _):
                row16 = k * _LANES + lanes

                def col(ref, ci):
                    return plsc.load_gather(
                        ref, [row16, jnp.full((_LANES,), ci, jnp.int32)])

                e16 = _pair_energy(
                    col(rows_i, 0), col(rows_i, 1), col(rows_i, 2),
                    col(rows_i, 3), col(rows_i, 4), col(rows_i, 5),
                    col(rows_i, 6),
                    col(rows_j, 0), col(rows_j, 1), col(rows_j, 2),
                    col(rows_j, 3), col(rows_j, 4), col(rows_j, 5),
                    col(rows_j, 6))
                evals[pl.ds(k * _LANES, _LANES)] = e16
                return 0

            lax.fori_loop(0, _CHUNK // _LANES, pair_body, 0)

            cps = []
            for sub in range(_NSUB):
                cps.append(pltpu.async_copy(
                    evals.at[pl.ds(sub * _SUBLEN, _SUBLEN)],
                    acc_sp.at[idxi_v.at[sub]], ssem, add=True))
            for cp in cps:
                cp.wait()

        load_idx(0, 0)
        gather_cps(0, True)

        def sched_body(g, _):
            load_idx(1, 2 * g + 1)
            gather_cps(1, True)
            for cp in gather_cps(0, False):
                cp.wait()
            compute_scatter(0)

            @pl.when(g < nchunk // 2 - 1)
            def _():
                load_idx(0, 2 * g + 2)
                gather_cps(0, True)

            for cp in gather_cps(1, False):
                cp.wait()
            compute_scatter(1)
            return 0

        lax.fori_loop(0, nchunk // 2, sched_body, 0)
        plsc.subcore_barrier()
        pltpu.sync_copy(acc_sp.at[pl.ds(r0, rows_per_tile)],
                        out_hbm.at[cid, tid])

    return sc_kernel


def _combine_body(p_ref, y_ref, o_ref):
    o_ref[...] = p_ref[0] + p_ref[1] + y_ref[...]


def kernel(yi, R, partial_charges, c6_table, Z, idx_m, idx_i, idx_j):
    n = Z.shape[0]
    p = idx_i.shape[0]

    nt = ((n + 1 + 2047) // 2048) * 2048
    pairs_per_sweep = _NW * _CHUNK
    nchunk = (p + pairs_per_sweep - 1) // pairs_per_sweep
    nchunk += nchunk % 2
    p_pad = nchunk * pairs_per_sweep

    Z = Z.astype(jnp.int32)
    zf = Z.astype(jnp.float32)
    c6 = jax.nn.softplus(c6_table.astype(jnp.float32))
    table = jnp.stack(
        [R[:, 0], R[:, 1], R[:, 2],
         partial_charges.astype(jnp.float32),
         c6[Z], zf, zf ** 0.23, jnp.zeros((n,), jnp.float32)], axis=1)
    table = jnp.pad(table, ((0, nt - n), (0, 0)))

    ii = jnp.pad(idx_i.astype(jnp.int32), (0, p_pad - p), constant_values=n)
    jj = jnp.pad(idx_j.astype(jnp.int32), (0, p_pad - p), constant_values=n)
    ii = ii.reshape(p_pad // _SUBLEN, _SUBLEN)
    jj = jj.reshape(p_pad // _SUBLEN, _SUBLEN)

    parts = _make_sc_kernel(nt, nchunk)(table, ii, jj)
    parts = parts.reshape(_NC, nt // 128, 128)

    yi0 = jnp.pad(yi[:, 0].astype(jnp.float32), (0, nt - n))
    yi0 = yi0.reshape(nt // 128, 128)

    total = pl.pallas_call(
        _combine_body,
        out_shape=jax.ShapeDtypeStruct((nt // 128, 128), jnp.float32),
    )(parts, yi0)

    return total.reshape(nt)[:n][:, None]


# speedup vs baseline: 332.6994x; 1.8519x over previous
import functools

import jax
import jax.numpy as jnp
from jax import lax
from jax.experimental import pallas as pl
from jax.experimental.pallas import tpu as pltpu
from jax.experimental.pallas import tpu_sc as plsc

_KE = 14.399645
_CUTON = 2.5
_SW_CUTOFF = 7.5
_LR_CUTOFF = 10.0
_CUTOFF = 10.0
_HALF_KE = 0.5 * _KE
_INV_SW_WIDTH = 1.0 / (_SW_CUTOFF - _CUTON)
_INV_LR = 1.0 / _LR_CUTOFF
_INV_LR2 = 1.0 / (_LR_CUTOFF * _LR_CUTOFF)
_INV_CUT = 1.0 / _CUTOFF
_INV_A_CONST = 1.0 / (0.8854 * 0.529177)

_NC = 2
_NS = 16
_NW = _NC * _NS
_LANES = 16
_SUBLEN = 128
_CHUNK = 1024
_NSUB = _CHUNK // _SUBLEN


def _rsqrt(x):
    i = lax.bitcast_convert_type(x, jnp.int32)
    i = jnp.int32(0x5F3759DF) - lax.shift_right_logical(i, 1)
    y = lax.bitcast_convert_type(i, jnp.float32)
    xh = 0.5 * x
    for _ in range(2):
        y = y * (1.5 - xh * y * y)
    return y


def _pair_energy(xi, yi_, zi_, qi, c6i, zfi, z23i,
                 xj, yj_, zj_, qj, c6j, zfj, z23j):
    dx = xj - xi
    dy = yj_ - yi_
    dz = zj_ - zi_
    r2 = dx * dx + dy * dy + dz * dz
    u = _rsqrt(r2)
    r = r2 * u
    damped = _rsqrt(r2 + 1.0)
    xs = jnp.clip((r - _CUTON) * _INV_SW_WIDTH, 0.0, 1.0)
    s = (xs * xs * xs) * (xs * (6.0 * xs - 15.0) + 10.0)
    coul = damped + s * (u - damped)
    in_lr = r < _LR_CUTOFF
    shifted = jnp.where(in_lr, coul + r * _INV_LR2 - 2.0 * _INV_LR, 0.0)
    e_c = (qi * qj) * shifted
    xc = r * _INV_CUT
    fc = jnp.where(in_lr,
                   1.0 - (xc * xc * xc) * (xc * (6.0 * xc - 15.0) + 10.0),
                   0.0)
    c6p = c6i * c6j + 1e-12
    c6ij = c6p * _rsqrt(c6p)
    r6 = r2 * r2 * r2
    w6 = _rsqrt(r6 + 1.0)
    e_d = (-0.5) * c6ij * (w6 * w6) * fc
    inv_a = (z23i + z23j + 1e-9) * _INV_A_CONST
    xz = r * inv_a
    phi = (0.18175 * jnp.exp(-3.19980 * xz)
           + 0.50986 * jnp.exp(-0.94229 * xz)
           + 0.28022 * jnp.exp(-0.40290 * xz)
           + 0.02817 * jnp.exp(-0.20162 * xz))
    e_z = (zfi * zfj) * u * phi * fc
    return _HALF_KE * (e_c + e_z) + e_d


def _make_sc_kernel(nt, nchunk):
    rows_per_tile = nt // _NS
    mesh = plsc.VectorSubcoreMesh(core_axis_name="c", subcore_axis_name="s",
                                  num_cores=_NC, num_subcores=_NS)

    scr = []
    for _ in range(2):
        scr += [
            pltpu.VMEM((_NSUB, _SUBLEN), jnp.int32),
            pltpu.VMEM((_NSUB, _SUBLEN), jnp.int32),
            pltpu.VMEM((_CHUNK, 8), jnp.float32),
            pltpu.VMEM((_CHUNK, 8), jnp.float32),
            pltpu.VMEM((_CHUNK,), jnp.float32),
            pltpu.SemaphoreType.DMA,
            pltpu.SemaphoreType.DMA,
        ]
    scr += [
        pltpu.VMEM((nt // _NS,), jnp.float32),
        pltpu.VMEM_SHARED((nt,), jnp.float32),
    ]

    @functools.partial(
        pl.kernel,
        out_type=jax.ShapeDtypeStruct((_NC, _NS, rows_per_tile), jnp.float32),
        mesh=mesh,
        scratch_types=scr,
        compiler_params=pltpu.CompilerParams(needs_layout_passes=False,
                                             use_tc_tiling_on_sc=False),
    )
    def sc_kernel(table_hbm, idxi_hbm, idxj_hbm, out_hbm, *bufs):
        (idxi_a, idxj_a, ri_a, rj_a, ev_a, sem_a, ssem_a,
         idxi_b, idxj_b, ri_b, rj_b, ev_b, sem_b, ssem_b,
         zbuf, acc_sp) = bufs
        buf = [(idxi_a, idxj_a, ri_a, rj_a, ev_a, sem_a, ssem_a),
               (idxi_b, idxj_b, ri_b, rj_b, ev_b, sem_b, ssem_b)]
        cid = lax.axis_index("c")
        tid = lax.axis_index("s")

        r0 = tid * rows_per_tile

        def zero_body(i, _):
            zbuf[pl.ds(i * _LANES, _LANES)] = jnp.zeros((_LANES,), jnp.float32)
            return 0
        lax.fori_loop(0, rows_per_tile // _LANES, zero_body, 0)
        pltpu.sync_copy(zbuf, acc_sp.at[pl.ds(r0, rows_per_tile)])
        plsc.subcore_barrier()

        wid = cid * _NS + tid
        base_row = wid * (nchunk * _NSUB)
        lanes = lax.iota(jnp.int32, _LANES)

        def load_idx(b, c):
            row0 = base_row + c * _NSUB
            sem = buf[b][5]
            c1 = pltpu.async_copy(idxi_hbm.at[pl.ds(row0, _NSUB)],
                                  buf[b][0], sem)
            c2 = pltpu.async_copy(idxj_hbm.at[pl.ds(row0, _NSUB)],
                                  buf[b][1], sem)
            c1.wait()
            c2.wait()

        def gather_cps(b, issue):
            idxi_v, idxj_v, rows_i, rows_j, _, sem, _ssem = buf[b]
            fn = pltpu.async_copy if issue else (
                lambda s, d, m: pltpu.make_async_copy(s, d, m))
            cps = []
            for sub in range(_NSUB):
                dst = pl.ds(sub * _SUBLEN, _SUBLEN)
                cps.append(fn(table_hbm.at[idxi_v.at[sub]],
                              rows_i.at[dst], sem))
                cps.append(fn(table_hbm.at[idxj_v.at[sub]],
                              rows_j.at[dst], sem))
            return cps

        def compute_scatter(b):
            idxi_v, _, rows_i, rows_j, evals, _sem, ssem = buf[b]

            def pair_body(k, _):
                row16 = k * _LANES + lanes

                def col(ref, ci):
                    return plsc.load_gather(
                        ref, [row16, jnp.full((_LANES,), ci, jnp.int32)])

                e16 = _pair_energy(
                    col(rows_i, 0), col(rows_i, 1), col(rows_i, 2),
                    col(rows_i, 3), col(rows_i, 4), col(rows_i, 5),
                    col(rows_i, 6),
                    col(rows_j, 0), col(rows_j, 1), col(rows_j, 2),
                    col(rows_j, 3), col(rows_j, 4), col(rows_j, 5),
                    col(rows_j, 6))
                evals[pl.ds(k * _LANES, _LANES)] = e16
                return 0

            lax.fori_loop(0, _CHUNK // _LANES, pair_body, 0)

            cps = []
            for sub in range(_NSUB):
                cps.append(pltpu.async_copy(
                    evals.at[pl.ds(sub * _SUBLEN, _SUBLEN)],
                    acc_sp.at[idxi_v.at[sub]], ssem, add=True))
            for cp in cps:
                cp.wait()

        load_idx(0, 0)
        gather_cps(0, True)

        def sched_body(g, _):
            load_idx(1, 2 * g + 1)
            gather_cps(1, True)
            for cp in gather_cps(0, False):
                cp.wait()
            compute_scatter(0)

            @pl.when(g < nchunk // 2 - 1)
            def _():
                load_idx(0, 2 * g + 2)
                gather_cps(0, True)

            for cp in gather_cps(1, False):
                cp.wait()
            compute_scatter(1)
            return 0

        lax.fori_loop(0, nchunk // 2, sched_body, 0)
        plsc.subcore_barrier()
        pltpu.sync_copy(acc_sp.at[pl.ds(r0, rows_per_tile)],
                        out_hbm.at[cid, tid])

    return sc_kernel


def _combine_body(p_ref, y_ref, o_ref):
    o_ref[...] = p_ref[0] + p_ref[1] + y_ref[...]


def kernel(yi, R, partial_charges, c6_table, Z, idx_m, idx_i, idx_j):
    n = Z.shape[0]
    p = idx_i.shape[0]

    nt = ((n + 1 + 2047) // 2048) * 2048
    pairs_per_sweep = _NW * _CHUNK
    nchunk = (p + pairs_per_sweep - 1) // pairs_per_sweep
    nchunk += nchunk % 2
    p_pad = nchunk * pairs_per_sweep

    Z = Z.astype(jnp.int32)
    zf = Z.astype(jnp.float32)
    c6 = jax.nn.softplus(c6_table.astype(jnp.float32))
    onehot = (Z[:, None] == jnp.arange(c6.shape[0], dtype=jnp.int32)[None, :])
    c6z = jnp.dot(onehot.astype(jnp.float32), c6)
    table = jnp.stack(
        [R[:, 0], R[:, 1], R[:, 2],
         partial_charges.astype(jnp.float32),
         c6z, zf, zf ** 0.23, jnp.zeros((n,), jnp.float32)], axis=1)
    table = jnp.pad(table, ((0, nt - n), (0, 0)))

    ii = jnp.pad(idx_i.astype(jnp.int32), (0, p_pad - p), constant_values=n)
    jj = jnp.pad(idx_j.astype(jnp.int32), (0, p_pad - p), constant_values=n)
    ii = ii.reshape(p_pad // _SUBLEN, _SUBLEN)
    jj = jj.reshape(p_pad // _SUBLEN, _SUBLEN)

    parts = _make_sc_kernel(nt, nchunk)(table, ii, jj)
    parts = parts.reshape(_NC, nt // 128, 128)

    yi0 = jnp.pad(yi[:, 0].astype(jnp.float32), (0, nt - n))
    yi0 = yi0.reshape(nt // 128, 128)

    total = pl.pallas_call(
        _combine_body,
        out_shape=jax.ShapeDtypeStruct((nt // 128, 128), jnp.float32),
    )(parts, yi0)

    return total.reshape(nt)[:n][:, None]
